# baseline (speedup 1.0000x reference)
import functools

import jax
import jax.numpy as jnp
from jax.experimental import pallas as pl
from jax.experimental.pallas import tpu as pltpu


def _linear_kernel(x_ref, w_hbm, b_hbm, o_ref, w_vmem, b_vmem, w_sem, b_sem):
    i = pl.program_id(0)

    @pl.when(i == 0)
    def _():
        w_cp = pltpu.make_async_copy(w_hbm, w_vmem, w_sem)
        b_cp = pltpu.make_async_copy(b_hbm, b_vmem, b_sem)
        w_cp.start()
        b_cp.start()
        w_cp.wait()
        b_cp.wait()

    acc = jnp.dot(x_ref[...], w_vmem[...],
                  preferred_element_type=jnp.float32) + b_vmem[...]
    o_ref[...] = acc[:, : o_ref.shape[1]]


@functools.partial(jax.jit, static_argnames=("output_size", "tm"))
def _forward(bow_vec, w_p, b_p, *, output_size, tm):
    B, F = bow_vec.shape
    F_pad, O_pad = w_p.shape

    w_hbm = pltpu.with_memory_space_constraint(w_p, pltpu.MemorySpace.HBM)
    b_hbm = pltpu.with_memory_space_constraint(b_p, pltpu.MemorySpace.HBM)
    return pl.pallas_call(
        _linear_kernel,
        out_shape=jax.ShapeDtypeStruct((B, output_size), jnp.float32),
        grid=(B // tm,),
        in_specs=[
            pl.BlockSpec((tm, F_pad), lambda i: (i, 0)),
            pl.BlockSpec(memory_space=pltpu.MemorySpace.HBM),
            pl.BlockSpec(memory_space=pltpu.MemorySpace.HBM),
        ],
        out_specs=pl.BlockSpec((tm, output_size), lambda i: (i, 0)),
        scratch_shapes=[
            pltpu.VMEM((F_pad, O_pad), jnp.float32),
            pltpu.VMEM((1, O_pad), jnp.float32),
            pltpu.SemaphoreType.DMA,
            pltpu.SemaphoreType.DMA,
        ],
        compiler_params=pltpu.CompilerParams(
            dimension_semantics=("arbitrary",),
            vmem_limit_bytes=48 * 1024 * 1024,
        ),
    )(bow_vec, w_hbm, b_hbm)


def kernel(bow_vec, w_p, b_p):
    return _forward(bow_vec, w_p, b_p, output_size=100, tm=512)


# speedup vs baseline: 1.1407x; 1.1407x over previous
import functools

import jax
import jax.numpy as jnp
from jax.experimental import pallas as pl
from jax.experimental.pallas import tpu as pltpu


def _linear_kernel(x_ref, w_ref, b_ref, o_ref):
    acc = jnp.dot(x_ref[...], w_ref[...],
                  preferred_element_type=jnp.float32) + b_ref[...]
    o_ref[...] = acc[:, : o_ref.shape[1]]


@functools.partial(jax.jit, static_argnames=("output_size", "tm"))
def _forward(bow_vec, w_p, b_p, *, output_size, tm):
    B, F = bow_vec.shape
    F_pad, O_pad = w_p.shape

    w_hbm = pltpu.with_memory_space_constraint(w_p, pltpu.MemorySpace.HBM)
    b_hbm = pltpu.with_memory_space_constraint(b_p, pltpu.MemorySpace.HBM)
    return pl.pallas_call(
        _linear_kernel,
        out_shape=jax.ShapeDtypeStruct((B, output_size), jnp.float32),
        grid=(B // tm,),
        in_specs=[
            pl.BlockSpec((tm, F_pad), lambda i: (i, 0)),
            pl.BlockSpec((F_pad, O_pad), lambda i: (0, 0)),
            pl.BlockSpec((1, O_pad), lambda i: (0, 0)),
        ],
        out_specs=pl.BlockSpec((tm, output_size), lambda i: (i, 0)),
        compiler_params=pltpu.CompilerParams(
            dimension_semantics=("arbitrary",),
            vmem_limit_bytes=48 * 1024 * 1024,
        ),
    )(bow_vec, w_hbm, b_hbm)


def kernel(bow_vec, w_p, b_p):
    return _forward(bow_vec, w_p, b_p, output_size=100, tm=512)
